# Initial kernel scaffold; baseline (speedup 1.0000x reference)
import functools

import jax
import jax.numpy as jnp
from jax import lax
from jax.experimental import pallas as pl
from jax.experimental.pallas import tpu as pltpu
from jax.experimental.pallas import tpu_sc as plsc

VOCAB = 12
EMB = 6
B, T = 4096, 200
N = B * T
NC, NS, L = 2, 16, 16
NW = NC * NS
PER_W = N // NW
CHUNK = 6400
NCHUNK = PER_W // CHUNK
TAB_PAD = 80


def _prep_body(tab_ref, ext_ref, p_ref):
    tab = tab_ref[...]
    m = jnp.max(tab, axis=1, keepdims=True)
    logz = m + jnp.log(jnp.sum(jnp.exp(tab - m), axis=1, keepdims=True))
    p_ref[...] = logz - ext_ref[...]


_prep = pl.pallas_call(
    _prep_body,
    out_shape=jax.ShapeDtypeStruct((VOCAB, VOCAB), jnp.float32),
)


def _fin_body(part_ref, loss_ref):
    loss_ref[0, 0] = jnp.sum(part_ref[...]) * jnp.float32(1.0 / N)


_finalize = pl.pallas_call(
    _fin_body,
    out_shape=jax.ShapeDtypeStruct((1, 1), jnp.float32),
    out_specs=pl.BlockSpec(memory_space=pltpu.SMEM),
)


_SC_MESH = plsc.VectorSubcoreMesh(
    core_axis_name="c", subcore_axis_name="s", num_cores=NC, num_subcores=NS
)


@functools.partial(
    pl.kernel,
    out_type=(
        jax.ShapeDtypeStruct((N * EMB,), jnp.float32),
        jax.ShapeDtypeStruct((NW, L), jnp.float32),
    ),
    mesh=_SC_MESH,
    scratch_types=(
        pltpu.VMEM((TAB_PAD,), jnp.float32),
        pltpu.VMEM((VOCAB * VOCAB,), jnp.float32),
        pltpu.VMEM((2, CHUNK), jnp.int32),
        pltpu.VMEM((2, CHUNK), jnp.int32),
        pltpu.VMEM((2, CHUNK * EMB), jnp.float32),
        pltpu.VMEM((L,), jnp.float32),
        pltpu.SemaphoreType.DMA,
        pltpu.SemaphoreType.DMA,
        pltpu.SemaphoreType.DMA,
        pltpu.SemaphoreType.DMA,
    ),
)
def _sc_main(in_hbm, tgt_hbm, tab_hbm, p_hbm, out_hbm, part_hbm,
             tab_v, p_v, in_v, tgt_v, out_v, acc_v,
             sem_in, sem_tgt, sem_out0, sem_out1):
    wid = lax.axis_index("s") * NC + lax.axis_index("c")
    base = wid * PER_W

    pltpu.sync_copy(tab_hbm, tab_v)
    pltpu.sync_copy(p_hbm, p_v)

    iota6 = lax.iota(jnp.int32, L) * 6
    sem_out = (sem_out0, sem_out1)

    def start_in(k):
        b = k & 1
        ci = pltpu.async_copy(
            in_hbm.at[pl.ds(base + k * CHUNK, CHUNK)], in_v.at[b], sem_in)
        ct = pltpu.async_copy(
            tgt_hbm.at[pl.ds(base + k * CHUNK, CHUNK)], tgt_v.at[b], sem_tgt)
        return ci, ct

    def compute_chunk(b, acc0):
        inb = in_v.at[b]
        tgb = tgt_v.at[b]
        onb = out_v.at[b]

        def body(i, acc):
            off = i * L
            vin = inb[pl.ds(off, L)]
            vtg = tgb[pl.ds(off, L)]
            b6 = vin * 6
            pid6 = iota6 + off * 6
            for c in range(EMB):
                val = plsc.load_gather(tab_v, [b6 + c])
                plsc.store_scatter(onb, [pid6 + c], val)
            return acc + plsc.load_gather(p_v, [b6 * 2 + vtg])

        return lax.fori_loop(0, CHUNK // L, body, acc0, unroll=2)

    pend = start_in(0)
    acc = jnp.zeros((L,), jnp.float32)
    out_pend = [None, None]
    for k in range(NCHUNK):
        b = k & 1
        ci, ct = pend
        ci.wait()
        ct.wait()
        if k + 1 < NCHUNK:
            pend = start_in(k + 1)
        if out_pend[b] is not None:
            out_pend[b].wait()
        acc = compute_chunk(b, acc)
        out_pend[b] = pltpu.async_copy(
            out_v.at[b],
            out_hbm.at[pl.ds((base + k * CHUNK) * EMB, CHUNK * EMB)],
            sem_out[b])
    for b in range(2):
        if out_pend[b] is not None:
            out_pend[b].wait()
    acc_v[...] = acc
    pltpu.sync_copy(acc_v, part_hbm.at[wid])


def kernel(inputs_BT, targets_BT, embedding_table):
    tab = embedding_table.astype(jnp.float32)
    inp = inputs_BT.reshape(-1).astype(jnp.int32)
    tgt = targets_BT.reshape(-1).astype(jnp.int32)
    tab_ext = jnp.concatenate(
        [tab, jnp.full((VOCAB, VOCAB - EMB), jnp.nan, jnp.float32)], axis=1)
    p_tab = _prep(tab, tab_ext)
    tab_flat = jnp.pad(tab.reshape(-1), (0, TAB_PAD - VOCAB * EMB))
    out_flat, part = _sc_main(inp, tgt, tab_flat, p_tab.reshape(-1))
    logits = out_flat.reshape(B, T, EMB)
    loss = _finalize(part)[0, 0]
    return logits, loss


# speedup vs baseline: 4.6943x; 4.6943x over previous
import functools

import jax
import jax.numpy as jnp
from jax import lax
from jax.experimental import pallas as pl
from jax.experimental.pallas import tpu as pltpu
from jax.experimental.pallas import tpu_sc as plsc

VOCAB = 12
EMB = 6
B, T = 4096, 200
N = B * T
NC, NS, L = 2, 16, 16
NW = NC * NS
PER_W = N // NW
CHUNK = 6400
NCHUNK = PER_W // CHUNK
TAB_PAD = 80


def _prep_body(tab_ref, ext_ref, p_ref):
    tab = tab_ref[...]
    m = jnp.max(tab, axis=1, keepdims=True)
    logz = m + jnp.log(jnp.sum(jnp.exp(tab - m), axis=1, keepdims=True))
    p_ref[...] = logz - ext_ref[...]


_prep = pl.pallas_call(
    _prep_body,
    out_shape=jax.ShapeDtypeStruct((VOCAB, VOCAB), jnp.float32),
)


def _fin_body(part_ref, loss_ref):
    loss_ref[0, 0] = jnp.sum(part_ref[...]) * jnp.float32(1.0 / N)


_finalize = pl.pallas_call(
    _fin_body,
    out_shape=jax.ShapeDtypeStruct((1, 1), jnp.float32),
    out_specs=pl.BlockSpec(memory_space=pltpu.SMEM),
)


_SC_MESH = plsc.VectorSubcoreMesh(
    core_axis_name="c", subcore_axis_name="s", num_cores=NC, num_subcores=NS
)


@functools.partial(
    pl.kernel,
    out_type=(
        jax.ShapeDtypeStruct((N * EMB,), jnp.float32),
        jax.ShapeDtypeStruct((NW, L), jnp.float32),
    ),
    mesh=_SC_MESH,
    compiler_params=pltpu.CompilerParams(needs_layout_passes=False),
    scratch_types=(
        pltpu.VMEM((TAB_PAD,), jnp.float32),
        pltpu.VMEM((VOCAB * VOCAB,), jnp.float32),
        pltpu.VMEM((CHUNK,), jnp.int32),
        pltpu.VMEM((CHUNK,), jnp.int32),
        pltpu.VMEM((CHUNK,), jnp.int32),
        pltpu.VMEM((CHUNK,), jnp.int32),
        pltpu.VMEM((CHUNK * EMB,), jnp.float32),
        pltpu.VMEM((CHUNK * EMB,), jnp.float32),
        pltpu.VMEM((L,), jnp.float32),
        pltpu.SemaphoreType.DMA,
        pltpu.SemaphoreType.DMA,
        pltpu.SemaphoreType.DMA,
        pltpu.SemaphoreType.DMA,
    ),
)
def _sc_main(in_hbm, tgt_hbm, tab_hbm, p_hbm, out_hbm, part_hbm,
             tab_v, p_v, in_v0, in_v1, tgt_v0, tgt_v1, out_v0, out_v1, acc_v,
             sem_in, sem_tgt, sem_out0, sem_out1):
    wid = lax.axis_index("s") * NC + lax.axis_index("c")
    base = wid * PER_W

    pltpu.sync_copy(tab_hbm, tab_v)
    pltpu.sync_copy(p_hbm, p_v)

    iota6 = lax.iota(jnp.int32, L) * 6
    in_v = (in_v0, in_v1)
    tgt_v = (tgt_v0, tgt_v1)
    out_v = (out_v0, out_v1)
    sem_out = (sem_out0, sem_out1)

    def start_in(k):
        b = k & 1
        ci = pltpu.async_copy(
            in_hbm.at[pl.ds(base + k * CHUNK, CHUNK)], in_v[b], sem_in)
        ct = pltpu.async_copy(
            tgt_hbm.at[pl.ds(base + k * CHUNK, CHUNK)], tgt_v[b], sem_tgt)
        return ci, ct

    def compute_chunk(b, acc0):
        inb = in_v[b]
        tgb = tgt_v[b]
        onb = out_v[b]

        def body(i, acc):
            off = i * L
            vin = inb[pl.ds(off, L)]
            vtg = tgb[pl.ds(off, L)]
            b6 = vin * 6
            pid6 = iota6 + off * 6
            for c in range(EMB):
                val = plsc.load_gather(tab_v, [b6 + c])
                plsc.store_scatter(onb, [pid6 + c], val)
            return acc + plsc.load_gather(p_v, [b6 * 2 + vtg])

        return lax.fori_loop(0, CHUNK // L, body, acc0, unroll=2)

    pend = start_in(0)
    acc = jnp.zeros((L,), jnp.float32)
    out_pend = [None, None]
    for k in range(NCHUNK):
        b = k & 1
        ci, ct = pend
        ci.wait()
        ct.wait()
        if k + 1 < NCHUNK:
            pend = start_in(k + 1)
        if out_pend[b] is not None:
            out_pend[b].wait()
        acc = compute_chunk(b, acc)
        out_pend[b] = pltpu.async_copy(
            out_v[b],
            out_hbm.at[pl.ds((base + k * CHUNK) * EMB, CHUNK * EMB)],
            sem_out[b])
    for b in range(2):
        if out_pend[b] is not None:
            out_pend[b].wait()
    acc_v[...] = acc
    pltpu.sync_copy(acc_v, part_hbm.at[wid])


def kernel(inputs_BT, targets_BT, embedding_table):
    tab = embedding_table.astype(jnp.float32)
    inp = inputs_BT.reshape(-1).astype(jnp.int32)
    tgt = targets_BT.reshape(-1).astype(jnp.int32)
    tab_ext = jnp.concatenate(
        [tab, jnp.full((VOCAB, VOCAB - EMB), jnp.nan, jnp.float32)], axis=1)
    p_tab = _prep(tab, tab_ext)
    tab_flat = jnp.pad(tab.reshape(-1), (0, TAB_PAD - VOCAB * EMB))
    out_flat, part = _sc_main(inp, tgt, tab_flat, p_tab.reshape(-1))
    logits = out_flat.reshape(B, T, EMB)
    loss = _finalize(part)[0, 0]
    return logits, loss


# speedup vs baseline: 32.4510x; 6.9128x over previous
import functools

import jax
import jax.numpy as jnp
from jax import lax
from jax.experimental import pallas as pl
from jax.experimental.pallas import tpu as pltpu
from jax.experimental.pallas import tpu_sc as plsc

VOCAB = 12
EMB = 6
B, T = 4096, 200
N = B * T
NC, NS, L = 2, 16, 16
NW = NC * NS
PER_W = N // NW
CHUNK = 6400
NCHUNK = PER_W // CHUNK
TAB_PAD = 80


def _prep_body(tab_ref, ext_ref, p_ref):
    tab = tab_ref[...]
    m = jnp.max(tab, axis=1, keepdims=True)
    logz = m + jnp.log(jnp.sum(jnp.exp(tab - m), axis=1, keepdims=True))
    p_ref[...] = logz - ext_ref[...]


_prep = pl.pallas_call(
    _prep_body,
    out_shape=jax.ShapeDtypeStruct((VOCAB, VOCAB), jnp.float32),
)


def _fin_body(part_ref, loss_ref):
    loss_ref[0, 0] = jnp.sum(part_ref[...]) * jnp.float32(1.0 / N)


_finalize = pl.pallas_call(
    _fin_body,
    out_shape=jax.ShapeDtypeStruct((1, 1), jnp.float32),
    out_specs=pl.BlockSpec(memory_space=pltpu.SMEM),
)


_SC_MESH = plsc.VectorSubcoreMesh(
    core_axis_name="c", subcore_axis_name="s", num_cores=NC, num_subcores=NS
)


@functools.partial(
    pl.kernel,
    out_type=(
        jax.ShapeDtypeStruct((N * EMB,), jnp.float32),
        jax.ShapeDtypeStruct((NW, L), jnp.float32),
    ),
    mesh=_SC_MESH,
    compiler_params=pltpu.CompilerParams(needs_layout_passes=False),
    scratch_types=(
        pltpu.VMEM((TAB_PAD,), jnp.float32),
        pltpu.VMEM((VOCAB * VOCAB,), jnp.float32),
        pltpu.VMEM((CHUNK,), jnp.int32),
        pltpu.VMEM((CHUNK,), jnp.int32),
        pltpu.VMEM((CHUNK,), jnp.int32),
        pltpu.VMEM((CHUNK,), jnp.int32),
        pltpu.VMEM((EMB * CHUNK,), jnp.float32),
        pltpu.VMEM((EMB * CHUNK,), jnp.float32),
        pltpu.VMEM((L,), jnp.float32),
        pltpu.SemaphoreType.DMA,
        pltpu.SemaphoreType.DMA,
        pltpu.SemaphoreType.DMA,
        pltpu.SemaphoreType.DMA,
    ),
)
def _sc_main(in_hbm, tgt_hbm, tab_hbm, p_hbm, out_hbm, part_hbm,
             tab_v, p_v, in_v0, in_v1, tgt_v0, tgt_v1, out_v0, out_v1, acc_v,
             sem_in, sem_tgt, sem_out0, sem_out1):
    wid = lax.axis_index("s") * NC + lax.axis_index("c")
    base = wid * PER_W

    pltpu.sync_copy(tab_hbm, tab_v)
    pltpu.sync_copy(p_hbm, p_v)

    in_v = (in_v0, in_v1)
    tgt_v = (tgt_v0, tgt_v1)
    out_v = (out_v0, out_v1)
    sem_out = (sem_out0, sem_out1)

    def start_in(k):
        b = k & 1
        ci = pltpu.async_copy(
            in_hbm.at[pl.ds(base + k * CHUNK, CHUNK)], in_v[b], sem_in)
        ct = pltpu.async_copy(
            tgt_hbm.at[pl.ds(base + k * CHUNK, CHUNK)], tgt_v[b], sem_tgt)
        return ci, ct

    def compute_chunk(b, acc0):
        inb = in_v[b]
        tgb = tgt_v[b]
        onb = out_v[b]

        def body(i, acc):
            off = i * L
            vin = inb[pl.ds(off, L)]
            vtg = tgb[pl.ds(off, L)]
            b6 = vin * 6
            for c in range(EMB):
                onb[pl.ds(c * CHUNK + off, L)] = plsc.load_gather(
                    tab_v, [b6 + c])
            return acc + plsc.load_gather(p_v, [b6 * 2 + vtg])

        return lax.fori_loop(0, CHUNK // L, body, acc0, unroll=2)

    pend = start_in(0)
    acc = jnp.zeros((L,), jnp.float32)
    out_pend = [None, None]
    for k in range(NCHUNK):
        b = k & 1
        ci, ct = pend
        ci.wait()
        ct.wait()
        if k + 1 < NCHUNK:
            pend = start_in(k + 1)
        if out_pend[b] is not None:
            for cp in out_pend[b]:
                cp.wait()
        acc = compute_chunk(b, acc)
        out_pend[b] = [
            pltpu.async_copy(
                out_v[b].at[pl.ds(c * CHUNK, CHUNK)],
                out_hbm.at[pl.ds(c * N + base + k * CHUNK, CHUNK)],
                sem_out[b])
            for c in range(EMB)
        ]
    for b in range(2):
        if out_pend[b] is not None:
            for cp in out_pend[b]:
                cp.wait()
    acc_v[...] = acc
    pltpu.sync_copy(acc_v, part_hbm.at[wid])


def kernel(inputs_BT, targets_BT, embedding_table):
    tab = embedding_table.astype(jnp.float32)
    inp = inputs_BT.astype(jnp.int32).T.reshape(-1)
    tgt = targets_BT.astype(jnp.int32).T.reshape(-1)
    tab_ext = jnp.concatenate(
        [tab, jnp.full((VOCAB, VOCAB - EMB), jnp.nan, jnp.float32)], axis=1)
    p_tab = _prep(tab, tab_ext)
    tab_flat = jnp.pad(tab.reshape(-1), (0, TAB_PAD - VOCAB * EMB))
    out_flat, part = _sc_main(inp, tgt, tab_flat, p_tab.reshape(-1))
    logits = out_flat.reshape(EMB, T, B).transpose(2, 1, 0)
    loss = _finalize(part)[0, 0]
    return logits, loss


# speedup vs baseline: 51.9747x; 1.6016x over previous
import functools

import jax
import jax.numpy as jnp
from jax import lax
from jax.experimental import pallas as pl
from jax.experimental.pallas import tpu as pltpu
from jax.experimental.pallas import tpu_sc as plsc

VOCAB = 12
EMB = 6
B, T = 4096, 200
N = B * T
NC, NS, L = 2, 16, 16
NW = NC * NS
PER_W = N // NW
CHUNK = 6400
NCHUNK = PER_W // CHUNK
TAB_PAD = 80


def _prep_body(tab_ref, ext_ref, p_ref):
    tab = tab_ref[...]
    m = jnp.max(tab, axis=1, keepdims=True)
    logz = m + jnp.log(jnp.sum(jnp.exp(tab - m), axis=1, keepdims=True))
    p_ref[...] = logz - ext_ref[...]


_prep = pl.pallas_call(
    _prep_body,
    out_shape=jax.ShapeDtypeStruct((VOCAB, VOCAB), jnp.float32),
)


def _fin_body(part_ref, loss_ref):
    loss_ref[0, 0] = jnp.sum(part_ref[...]) * jnp.float32(1.0 / N)


_finalize = pl.pallas_call(
    _fin_body,
    out_shape=jax.ShapeDtypeStruct((1, 1), jnp.float32),
    out_specs=pl.BlockSpec(memory_space=pltpu.SMEM),
)


_SC_MESH = plsc.VectorSubcoreMesh(
    core_axis_name="c", subcore_axis_name="s", num_cores=NC, num_subcores=NS
)


@functools.partial(
    pl.kernel,
    out_type=(
        jax.ShapeDtypeStruct((N * EMB,), jnp.float32),
        jax.ShapeDtypeStruct((NW, L), jnp.float32),
    ),
    mesh=_SC_MESH,
    compiler_params=pltpu.CompilerParams(needs_layout_passes=False),
    scratch_types=(
        pltpu.VMEM((TAB_PAD,), jnp.float32),
        pltpu.VMEM((VOCAB * VOCAB,), jnp.float32),
        pltpu.VMEM((CHUNK,), jnp.int32),
        pltpu.VMEM((CHUNK,), jnp.int32),
        pltpu.VMEM((CHUNK,), jnp.int32),
        pltpu.VMEM((CHUNK,), jnp.int32),
        pltpu.VMEM((EMB * CHUNK,), jnp.float32),
        pltpu.VMEM((EMB * CHUNK,), jnp.float32),
        pltpu.VMEM((L,), jnp.float32),
        pltpu.SemaphoreType.DMA,
        pltpu.SemaphoreType.DMA,
        pltpu.SemaphoreType.DMA,
        pltpu.SemaphoreType.DMA,
    ),
)
def _sc_main(in_hbm, tgt_hbm, tab_hbm, p_hbm, out_hbm, part_hbm,
             tab_v, p_v, in_v0, in_v1, tgt_v0, tgt_v1, out_v0, out_v1, acc_v,
             sem_in, sem_tgt, sem_out0, sem_out1):
    wid = lax.axis_index("s") * NC + lax.axis_index("c")
    base = wid * PER_W

    pltpu.sync_copy(tab_hbm, tab_v)
    pltpu.sync_copy(p_hbm, p_v)

    in_v = (in_v0, in_v1)
    tgt_v = (tgt_v0, tgt_v1)
    out_v = (out_v0, out_v1)
    sem_out = (sem_out0, sem_out1)

    def start_in(k):
        b = k & 1
        ci = pltpu.async_copy(
            in_hbm.at[pl.ds(base + k * CHUNK, CHUNK)], in_v[b], sem_in)
        ct = pltpu.async_copy(
            tgt_hbm.at[pl.ds(base + k * CHUNK, CHUNK)], tgt_v[b], sem_tgt)
        return ci, ct

    def compute_chunk(b, acc0):
        inb = in_v[b]
        tgb = tgt_v[b]
        onb = out_v[b]

        @plsc.parallel_loop(0, CHUNK, step=L, unroll=4, carry=acc0)
        def body(off, acc):
            vin = inb[pl.ds(off, L)]
            vtg = tgb[pl.ds(off, L)]
            b6 = vin * 6
            for c in range(EMB):
                onb[pl.ds(c * CHUNK + off, L)] = plsc.load_gather(
                    tab_v, [b6 + c])
            return acc + plsc.load_gather(p_v, [b6 * 2 + vtg])

        return body

    pend = start_in(0)
    acc = jnp.zeros((L,), jnp.float32)
    out_pend = [None, None]
    for k in range(NCHUNK):
        b = k & 1
        ci, ct = pend
        ci.wait()
        ct.wait()
        if k + 1 < NCHUNK:
            pend = start_in(k + 1)
        if out_pend[b] is not None:
            for cp in out_pend[b]:
                cp.wait()
        acc = compute_chunk(b, acc)
        out_pend[b] = [
            pltpu.async_copy(
                out_v[b].at[pl.ds(c * CHUNK, CHUNK)],
                out_hbm.at[pl.ds(c * N + base + k * CHUNK, CHUNK)],
                sem_out[b])
            for c in range(EMB)
        ]
    for b in range(2):
        if out_pend[b] is not None:
            for cp in out_pend[b]:
                cp.wait()
    acc_v[...] = acc
    pltpu.sync_copy(acc_v, part_hbm.at[wid])


def kernel(inputs_BT, targets_BT, embedding_table):
    tab = embedding_table.astype(jnp.float32)
    inp = inputs_BT.astype(jnp.int32).T.reshape(-1)
    tgt = targets_BT.astype(jnp.int32).T.reshape(-1)
    tab_ext = jnp.concatenate(
        [tab, jnp.full((VOCAB, VOCAB - EMB), jnp.nan, jnp.float32)], axis=1)
    p_tab = _prep(tab, tab_ext)
    tab_flat = jnp.pad(tab.reshape(-1), (0, TAB_PAD - VOCAB * EMB))
    out_flat, part = _sc_main(inp, tgt, tab_flat, p_tab.reshape(-1))
    logits = out_flat.reshape(EMB, T, B).transpose(2, 1, 0)
    loss = _finalize(part)[0, 0]
    return logits, loss


# speedup vs baseline: 52.0642x; 1.0017x over previous
import functools

import jax
import jax.numpy as jnp
from jax import lax
from jax.experimental import pallas as pl
from jax.experimental.pallas import tpu as pltpu
from jax.experimental.pallas import tpu_sc as plsc

VOCAB = 12
EMB = 6
B, T = 4096, 200
N = B * T
NC, NS, L = 2, 16, 16
NW = NC * NS
PER_W = N // NW
CHUNK = 6400
NCHUNK = PER_W // CHUNK
TAB_PAD = 80


def _prep_body(tab_ref, ext_ref, p_ref):
    tab = tab_ref[...]
    m = jnp.max(tab, axis=1, keepdims=True)
    logz = m + jnp.log(jnp.sum(jnp.exp(tab - m), axis=1, keepdims=True))
    p_ref[...] = logz - ext_ref[...]


_prep = pl.pallas_call(
    _prep_body,
    out_shape=jax.ShapeDtypeStruct((VOCAB, VOCAB), jnp.float32),
)


def _fin_body(part_ref, loss_ref):
    loss_ref[0, 0] = jnp.sum(part_ref[...]) * jnp.float32(1.0 / N)


_finalize = pl.pallas_call(
    _fin_body,
    out_shape=jax.ShapeDtypeStruct((1, 1), jnp.float32),
    out_specs=pl.BlockSpec(memory_space=pltpu.SMEM),
)


_SC_MESH = plsc.VectorSubcoreMesh(
    core_axis_name="c", subcore_axis_name="s", num_cores=NC, num_subcores=NS
)


@functools.partial(
    pl.kernel,
    out_type=(
        jax.ShapeDtypeStruct((N * EMB,), jnp.float32),
        jax.ShapeDtypeStruct((NW, L), jnp.float32),
    ),
    mesh=_SC_MESH,
    compiler_params=pltpu.CompilerParams(needs_layout_passes=False),
    scratch_types=(
        pltpu.VMEM((TAB_PAD,), jnp.float32),
        pltpu.VMEM((VOCAB * VOCAB,), jnp.float32),
        pltpu.VMEM((CHUNK,), jnp.int32),
        pltpu.VMEM((CHUNK,), jnp.int32),
        pltpu.VMEM((CHUNK,), jnp.int32),
        pltpu.VMEM((CHUNK,), jnp.int32),
        pltpu.VMEM((EMB * CHUNK,), jnp.float32),
        pltpu.VMEM((EMB * CHUNK,), jnp.float32),
        pltpu.VMEM((L,), jnp.float32),
        pltpu.SemaphoreType.DMA,
        pltpu.SemaphoreType.DMA,
        pltpu.SemaphoreType.DMA,
        pltpu.SemaphoreType.DMA,
    ),
)
def _sc_main(in_hbm, tgt_hbm, tab_hbm, p_hbm, out_hbm, part_hbm,
             tab_v, p_v, in_v0, in_v1, tgt_v0, tgt_v1, out_v0, out_v1, acc_v,
             sem_in, sem_tgt, sem_out0, sem_out1):
    wid = lax.axis_index("s") * NC + lax.axis_index("c")
    base = wid * PER_W

    pltpu.sync_copy(tab_hbm, tab_v)
    pltpu.sync_copy(p_hbm, p_v)

    in_v = (in_v0, in_v1)
    tgt_v = (tgt_v0, tgt_v1)
    out_v = (out_v0, out_v1)
    sem_out = (sem_out0, sem_out1)

    def start_in(k):
        b = k & 1
        ci = pltpu.async_copy(
            in_hbm.at[pl.ds(base + k * CHUNK, CHUNK)], in_v[b], sem_in)
        ct = pltpu.async_copy(
            tgt_hbm.at[pl.ds(base + k * CHUNK, CHUNK)], tgt_v[b], sem_tgt)
        return ci, ct

    def compute_chunk(b, acc0):
        inb = in_v[b]
        tgb = tgt_v[b]
        onb = out_v[b]

        @plsc.parallel_loop(0, CHUNK, step=L, unroll=8, carry=acc0)
        def body(off, acc):
            vin = inb[pl.ds(off, L)]
            vtg = tgb[pl.ds(off, L)]
            b6 = vin * 6
            for c in range(EMB):
                onb[pl.ds(c * CHUNK + off, L)] = plsc.load_gather(
                    tab_v, [b6 + c])
            return acc + plsc.load_gather(p_v, [b6 * 2 + vtg])

        return body

    pend = start_in(0)
    acc = jnp.zeros((L,), jnp.float32)
    out_pend = [None, None]
    for k in range(NCHUNK):
        b = k & 1
        ci, ct = pend
        ci.wait()
        ct.wait()
        if k + 1 < NCHUNK:
            pend = start_in(k + 1)
        if out_pend[b] is not None:
            for cp in out_pend[b]:
                cp.wait()
        acc = compute_chunk(b, acc)
        out_pend[b] = [
            pltpu.async_copy(
                out_v[b].at[pl.ds(c * CHUNK, CHUNK)],
                out_hbm.at[pl.ds(c * N + base + k * CHUNK, CHUNK)],
                sem_out[b])
            for c in range(EMB)
        ]
    for b in range(2):
        if out_pend[b] is not None:
            for cp in out_pend[b]:
                cp.wait()
    acc_v[...] = acc
    pltpu.sync_copy(acc_v, part_hbm.at[wid])


def kernel(inputs_BT, targets_BT, embedding_table):
    tab = embedding_table.astype(jnp.float32)
    inp = inputs_BT.astype(jnp.int32).T.reshape(-1)
    tgt = targets_BT.astype(jnp.int32).T.reshape(-1)
    tab_ext = jnp.concatenate(
        [tab, jnp.full((VOCAB, VOCAB - EMB), jnp.nan, jnp.float32)], axis=1)
    p_tab = _prep(tab, tab_ext)
    tab_flat = jnp.pad(tab.reshape(-1), (0, TAB_PAD - VOCAB * EMB))
    out_flat, part = _sc_main(inp, tgt, tab_flat, p_tab.reshape(-1))
    logits = out_flat.reshape(EMB, T, B).transpose(2, 1, 0)
    loss = _finalize(part)[0, 0]
    return logits, loss


# speedup vs baseline: 90.2102x; 1.7327x over previous
import functools

import jax
import jax.numpy as jnp
from jax import lax
from jax.experimental import pallas as pl
from jax.experimental.pallas import tpu as pltpu
from jax.experimental.pallas import tpu_sc as plsc

VOCAB = 12
EMB = 6
B, T = 4096, 200
N = B * T
NC, NS, L = 2, 16, 16
NW = NC * NS
PER_W = N // NW
CHUNK = 6400
NCHUNK = PER_W // CHUNK
TAB_PAD = 80


def _prep_body(tab_ref, ext_ref, p_ref):
    tab = tab_ref[...]
    m = jnp.max(tab, axis=1, keepdims=True)
    logz = m + jnp.log(jnp.sum(jnp.exp(tab - m), axis=1, keepdims=True))
    p_ref[...] = logz - ext_ref[...]


_prep = pl.pallas_call(
    _prep_body,
    out_shape=jax.ShapeDtypeStruct((VOCAB, VOCAB), jnp.float32),
)


def _fin_body(part_ref, loss_ref):
    loss_ref[0, 0] = jnp.sum(part_ref[...]) * jnp.float32(1.0 / N)


_finalize = pl.pallas_call(
    _fin_body,
    out_shape=jax.ShapeDtypeStruct((1, 1), jnp.float32),
    out_specs=pl.BlockSpec(memory_space=pltpu.SMEM),
)


_SC_MESH = plsc.VectorSubcoreMesh(
    core_axis_name="c", subcore_axis_name="s", num_cores=NC, num_subcores=NS
)


@functools.partial(
    pl.kernel,
    out_type=(
        jax.ShapeDtypeStruct((N * EMB,), jnp.float32),
        jax.ShapeDtypeStruct((NW, L), jnp.float32),
    ),
    mesh=_SC_MESH,
    compiler_params=pltpu.CompilerParams(needs_layout_passes=False),
    scratch_types=(
        pltpu.VMEM((TAB_PAD,), jnp.float32),
        pltpu.VMEM((VOCAB * VOCAB,), jnp.float32),
        pltpu.VMEM((CHUNK,), jnp.int32),
        pltpu.VMEM((CHUNK,), jnp.int32),
        pltpu.VMEM((CHUNK,), jnp.int32),
        pltpu.VMEM((CHUNK,), jnp.int32),
        pltpu.VMEM((EMB * CHUNK,), jnp.float32),
        pltpu.VMEM((EMB * CHUNK,), jnp.float32),
        pltpu.VMEM((L,), jnp.float32),
        pltpu.SemaphoreType.DMA,
        pltpu.SemaphoreType.DMA,
        pltpu.SemaphoreType.DMA,
        pltpu.SemaphoreType.DMA,
    ),
)
def _sc_main(in_hbm, tgt_hbm, tab_hbm, p_hbm, out_hbm, part_hbm,
             tab_v, p_v, in_v0, in_v1, tgt_v0, tgt_v1, out_v0, out_v1, acc_v,
             sem_in, sem_tgt, sem_out0, sem_out1):
    wid = lax.axis_index("s") * NC + lax.axis_index("c")
    base = wid * PER_W

    pltpu.sync_copy(tab_hbm, tab_v)
    pltpu.sync_copy(p_hbm, p_v)

    in_v = (in_v0, in_v1)
    tgt_v = (tgt_v0, tgt_v1)
    out_v = (out_v0, out_v1)
    sem_out = (sem_out0, sem_out1)

    def start_in(k):
        b = k & 1
        ci = pltpu.async_copy(
            in_hbm.at[pl.ds(base + k * CHUNK, CHUNK)], in_v[b], sem_in)
        ct = pltpu.async_copy(
            tgt_hbm.at[pl.ds(base + k * CHUNK, CHUNK)], tgt_v[b], sem_tgt)
        return ci, ct

    def compute_chunk(b, acc0):
        inb = in_v[b]
        tgb = tgt_v[b]
        onb = out_v[b]

        @plsc.parallel_loop(0, CHUNK, step=L, unroll=8, carry=acc0)
        def body(off, acc):
            vin = inb[pl.ds(off, L)]
            vtg = tgb[pl.ds(off, L)]
            b6 = vin * 6
            for c in range(EMB):
                onb[pl.ds(c * CHUNK + off, L)] = plsc.load_gather(
                    tab_v, [b6 + c])
            return acc + plsc.load_gather(p_v, [b6 * 2 + vtg])

        return body

    pend = start_in(0)
    acc = jnp.zeros((L,), jnp.float32)
    out_pend = [None, None]
    for k in range(NCHUNK):
        b = k & 1
        ci, ct = pend
        ci.wait()
        ct.wait()
        if k + 1 < NCHUNK:
            pend = start_in(k + 1)
        if out_pend[b] is not None:
            for cp in out_pend[b]:
                cp.wait()
        acc = compute_chunk(b, acc)
        out_pend[b] = [
            pltpu.async_copy(
                out_v[b].at[pl.ds(c * CHUNK, CHUNK)],
                out_hbm.at[pl.ds(c * N + base + k * CHUNK, CHUNK)],
                sem_out[b])
            for c in range(EMB)
        ]
    for b in range(2):
        if out_pend[b] is not None:
            for cp in out_pend[b]:
                cp.wait()
    acc_v[...] = acc
    pltpu.sync_copy(acc_v, part_hbm.at[wid])


def _flatten_physical(x_BT):
    return (x_BT.astype(jnp.int32).T
            .reshape(T // 8, 8, B // 128, 128)
            .transpose(0, 2, 1, 3)
            .reshape(-1))


def kernel(inputs_BT, targets_BT, embedding_table):
    tab = embedding_table.astype(jnp.float32)
    inp = _flatten_physical(inputs_BT)
    tgt = _flatten_physical(targets_BT)
    tab_ext = jnp.concatenate(
        [tab, jnp.full((VOCAB, VOCAB - EMB), jnp.nan, jnp.float32)], axis=1)
    p_tab = _prep(tab, tab_ext)
    tab_flat = jnp.pad(tab.reshape(-1), (0, TAB_PAD - VOCAB * EMB))
    out_flat, part = _sc_main(inp, tgt, tab_flat, p_tab.reshape(-1))
    logits = (out_flat
              .reshape(EMB, T // 8, B // 128, 8, 128)
              .transpose(0, 1, 3, 2, 4)
              .reshape(EMB, T, B)
              .transpose(2, 1, 0))
    loss = _finalize(part)[0, 0]
    return logits, loss


# speedup vs baseline: 91.8188x; 1.0178x over previous
import functools

import jax
import jax.numpy as jnp
from jax import lax
from jax.experimental import pallas as pl
from jax.experimental.pallas import tpu as pltpu
from jax.experimental.pallas import tpu_sc as plsc

VOCAB = 12
EMB = 6
B, T = 4096, 200
N = B * T
NC, NS, L = 2, 16, 16
NW = NC * NS
PER_W = N // NW
CHUNK = 6400
NCHUNK = PER_W // CHUNK
TAB_PAD = 80


def _prep_body(tab_ref, ext_ref, p_ref):
    tab = tab_ref[...]
    m = jnp.max(tab, axis=1, keepdims=True)
    logz = m + jnp.log(jnp.sum(jnp.exp(tab - m), axis=1, keepdims=True))
    p_ref[...] = logz - ext_ref[...]


_prep = pl.pallas_call(
    _prep_body,
    out_shape=jax.ShapeDtypeStruct((VOCAB, VOCAB), jnp.float32),
)


def _fin_body(part_ref, loss_ref):
    loss_ref[0, 0] = jnp.sum(part_ref[...]) * jnp.float32(1.0 / N)


_finalize = pl.pallas_call(
    _fin_body,
    out_shape=jax.ShapeDtypeStruct((1, 1), jnp.float32),
    out_specs=pl.BlockSpec(memory_space=pltpu.SMEM),
)


_SC_MESH = plsc.VectorSubcoreMesh(
    core_axis_name="c", subcore_axis_name="s", num_cores=NC, num_subcores=NS
)


@functools.partial(
    pl.kernel,
    out_type=(
        jax.ShapeDtypeStruct((N * EMB,), jnp.float32),
        jax.ShapeDtypeStruct((NW, L), jnp.float32),
    ),
    mesh=_SC_MESH,
    compiler_params=pltpu.CompilerParams(needs_layout_passes=False),
    scratch_types=(
        pltpu.VMEM((TAB_PAD,), jnp.float32),
        pltpu.VMEM((VOCAB * VOCAB,), jnp.float32),
        pltpu.VMEM((CHUNK,), jnp.int32),
        pltpu.VMEM((CHUNK,), jnp.int32),
        pltpu.VMEM((CHUNK,), jnp.int32),
        pltpu.VMEM((CHUNK,), jnp.int32),
        pltpu.VMEM((EMB * CHUNK,), jnp.float32),
        pltpu.VMEM((EMB * CHUNK,), jnp.float32),
        pltpu.VMEM((L,), jnp.float32),
        pltpu.SemaphoreType.DMA,
        pltpu.SemaphoreType.DMA,
        pltpu.SemaphoreType.DMA,
        pltpu.SemaphoreType.DMA,
    ),
)
def _sc_main(in_hbm, tgt_hbm, tab_hbm, p_hbm, out_hbm, part_hbm,
             tab_v, p_v, in_v0, in_v1, tgt_v0, tgt_v1, out_v0, out_v1, acc_v,
             sem_in, sem_tgt, sem_out0, sem_out1):
    wid = lax.axis_index("s") * NC + lax.axis_index("c")
    base = wid * PER_W

    in_v = (in_v0, in_v1)
    tgt_v = (tgt_v0, tgt_v1)
    out_v = (out_v0, out_v1)
    sem_out = (sem_out0, sem_out1)

    def start_in(k):
        b = k & 1
        ci = pltpu.async_copy(
            in_hbm.at[pl.ds(base + k * CHUNK, CHUNK)], in_v[b], sem_in)
        ct = pltpu.async_copy(
            tgt_hbm.at[pl.ds(base + k * CHUNK, CHUNK)], tgt_v[b], sem_tgt)
        return ci, ct

    def compute_chunk(b, acc0):
        inb = in_v[b]
        tgb = tgt_v[b]
        onb = out_v[b]

        @plsc.parallel_loop(0, CHUNK, step=L, unroll=8, carry=acc0)
        def body(off, acc):
            vin = inb[pl.ds(off, L)]
            vtg = tgb[pl.ds(off, L)]
            b6 = vin * 6
            for c in range(EMB):
                onb[pl.ds(c * CHUNK + off, L)] = plsc.load_gather(
                    tab_v, [b6 + c])
            return acc + plsc.load_gather(p_v, [b6 * 2 + vtg])

        return body

    pend = start_in(0)
    pltpu.sync_copy(tab_hbm, tab_v)
    pltpu.sync_copy(p_hbm, p_v)
    acc = jnp.zeros((L,), jnp.float32)
    out_pend = [None, None]
    for k in range(NCHUNK):
        b = k & 1
        ci, ct = pend
        ci.wait()
        ct.wait()
        if k + 1 < NCHUNK:
            pend = start_in(k + 1)
        if out_pend[b] is not None:
            for cp in out_pend[b]:
                cp.wait()
        acc = compute_chunk(b, acc)
        out_pend[b] = [
            pltpu.async_copy(
                out_v[b].at[pl.ds(c * CHUNK, CHUNK)],
                out_hbm.at[pl.ds(c * N + base + k * CHUNK, CHUNK)],
                sem_out[b])
            for c in range(EMB)
        ]
    for b in range(2):
        if out_pend[b] is not None:
            for cp in out_pend[b]:
                cp.wait()
    acc_v[...] = acc
    pltpu.sync_copy(acc_v, part_hbm.at[wid])


def _flatten_physical(x_BT):
    return (x_BT.astype(jnp.int32).T
            .reshape(T // 8, 8, B // 128, 128)
            .transpose(0, 2, 1, 3)
            .reshape(-1))


def kernel(inputs_BT, targets_BT, embedding_table):
    tab = embedding_table.astype(jnp.float32)
    inp = _flatten_physical(inputs_BT)
    tgt = _flatten_physical(targets_BT)
    tab_ext = jnp.concatenate(
        [tab, jnp.full((VOCAB, VOCAB - EMB), jnp.nan, jnp.float32)], axis=1)
    p_tab = _prep(tab, tab_ext)
    tab_flat = jnp.pad(tab.reshape(-1), (0, TAB_PAD - VOCAB * EMB))
    out_flat, part = _sc_main(inp, tgt, tab_flat, p_tab.reshape(-1))
    logits = (out_flat
              .reshape(EMB, T // 8, B // 128, 8, 128)
              .transpose(0, 1, 3, 2, 4)
              .reshape(EMB, T, B)
              .transpose(2, 1, 0))
    loss = _finalize(part)[0, 0]
    return logits, loss
